# Initial kernel scaffold; baseline (speedup 1.0000x reference)
import jax
import jax.numpy as jnp
from jax.experimental import pallas as pl


def kernel(x, edge_index, W1, b1, W2, b2):
    raise NotImplementedError("write your pallas kernel here")


# speedup vs baseline: 44.3681x; 44.3681x over previous
import functools

import jax
import jax.numpy as jnp
from jax import lax
from jax.experimental import pallas as pl
from jax.experimental.pallas import tpu as pltpu
from jax.experimental.pallas import tpu_sc as plsc

N = 10000
E = 320000
F_IN = 128
HID = 8
OUT = 2

NC = 2
NS = 16
C = 128
CH = 80
R = 4
NG = CH // R
EPT = CH * C
EPAD = NC * NS * EPT
NP = 10240
RPT = NP // NS
BLK = 1024


def _mesh():
    return plsc.VectorSubcoreMesh(
        core_axis_name="c", subcore_axis_name="s",
        num_cores=NC, num_subcores=NS)


@functools.partial(
    pl.kernel,
    out_type=jax.ShapeDtypeStruct((NC, NP), jnp.float32),
    mesh=_mesh(),
    compiler_params=pltpu.CompilerParams(use_tc_tiling_on_sc=False),
    scratch_types=[
        pltpu.VMEM((CH, C), jnp.int32),
        pltpu.VMEM((C,), jnp.float32),
        pltpu.VMEM((RPT,), jnp.float32),
        pltpu.VMEM_SHARED((NP,), jnp.float32),
    ],
)
def _sc_degree(dstp, onesc, zrow, out, idx_v, ones_v, row_v, acc_sh):
    c = lax.axis_index("c")
    s = lax.axis_index("s")
    pltpu.sync_copy(dstp.at[c, s], idx_v)
    pltpu.sync_copy(onesc, ones_v)
    pltpu.sync_copy(zrow, row_v)
    pltpu.sync_copy(row_v, acc_sh.at[pl.ds(s * RPT, RPT)])
    plsc.subcore_barrier()

    def body(j, carry):
        pltpu.sync_copy(ones_v, acc_sh.at[idx_v.at[j]], add=True)
        return carry

    lax.fori_loop(0, CH, body, 0)
    plsc.subcore_barrier()
    pltpu.sync_copy(acc_sh.at[pl.ds(s * RPT, RPT)], row_v)
    pltpu.sync_copy(row_v, out.at[c, pl.ds(s * RPT, RPT)])


def _make_agg(D):
    @functools.partial(
        pl.kernel,
        out_type=jax.ShapeDtypeStruct((NC, NP, D), jnp.float32),
        mesh=_mesh(),
        compiler_params=pltpu.CompilerParams(use_tc_tiling_on_sc=False),
        scratch_types=[
            pltpu.VMEM((CH, C), jnp.int32),
            pltpu.VMEM((CH, C), jnp.int32),
            pltpu.VMEM((R, C, D), jnp.float32),
            pltpu.VMEM((RPT, D), jnp.float32),
            pltpu.VMEM_SHARED((NP, D), jnp.float32),
            pltpu.SemaphoreType.DMA,
            pltpu.SemaphoreType.DMA,
            pltpu.SemaphoreType.DMA,
            pltpu.SemaphoreType.DMA,
        ],
    )
    def agg(srcp, dstp, g, zrow, out,
            src_v, dst_v, rows_v, buf_v, acc_sh, s0, s1, s2, s3):
        sems = (s0, s1, s2, s3)
        c = lax.axis_index("c")
        s = lax.axis_index("s")
        pltpu.sync_copy(srcp.at[c, s], src_v)
        pltpu.sync_copy(dstp.at[c, s], dst_v)
        for b in range(R):
            pltpu.async_copy(g.at[src_v.at[b]], rows_v.at[b], sems[b])
        pltpu.sync_copy(zrow, buf_v)
        pltpu.sync_copy(buf_v, acc_sh.at[pl.ds(s * RPT, RPT)])
        plsc.subcore_barrier()

        def body(gi, carry):
            for b in range(R):
                j = gi * R + b
                pltpu.make_async_copy(
                    g.at[src_v.at[b]], rows_v.at[b], sems[b]).wait()
                pltpu.sync_copy(rows_v.at[b], acc_sh.at[dst_v.at[j]],
                                add=True)
                pltpu.async_copy(g.at[src_v.at[j + R]], rows_v.at[b],
                                 sems[b])
            return carry

        lax.fori_loop(0, NG - 1, body, 0)
        for b in range(R):
            j = (NG - 1) * R + b
            pltpu.make_async_copy(
                g.at[src_v.at[b]], rows_v.at[b], sems[b]).wait()
            pltpu.sync_copy(rows_v.at[b], acc_sh.at[dst_v.at[j]], add=True)
        plsc.subcore_barrier()
        pltpu.sync_copy(acc_sh.at[pl.ds(s * RPT, RPT)], buf_v)
        pltpu.sync_copy(buf_v, out.at[c, pl.ds(s * RPT, RPT)])

    return agg


_sc_agg_hid = _make_agg(HID)


def _tc1_body(xp_ref, w1_ref, degp_ref, g1_ref, dinv_ref):
    deg = degp_ref[0, :] + degp_ref[1, :] + 1.0
    dinv = lax.rsqrt(deg)[:, None]
    h = jnp.dot(xp_ref[...], w1_ref[...],
                preferred_element_type=jnp.float32)
    g1_ref[...] = h * dinv
    dinv_ref[...] = dinv


def _tc1(xp, w1, degp):
    return pl.pallas_call(
        _tc1_body,
        grid=(NP // BLK,),
        in_specs=[
            pl.BlockSpec((BLK, F_IN), lambda i: (i, 0)),
            pl.BlockSpec((F_IN, HID), lambda i: (0, 0)),
            pl.BlockSpec((NC, BLK), lambda i: (0, i)),
        ],
        out_specs=[
            pl.BlockSpec((BLK, HID), lambda i: (i, 0)),
            pl.BlockSpec((BLK, 1), lambda i: (i, 0)),
        ],
        out_shape=[
            jax.ShapeDtypeStruct((NP, HID), jnp.float32),
            jax.ShapeDtypeStruct((NP, 1), jnp.float32),
        ],
    )(xp, w1, degp)


def _tc2_body(p1_ref, g1_ref, dinv_ref, b1_ref, w2_ref, g2_ref):
    ssum = p1_ref[0] + p1_ref[1] + g1_ref[...]
    h = jnp.maximum(ssum * dinv_ref[...] + b1_ref[...], 0.0)
    h2 = jnp.dot(h, w2_ref[...], preferred_element_type=jnp.float32)
    g2_ref[...] = h2 * dinv_ref[...]


def _tc2(p1, g1, dinv, b1, w2):
    return pl.pallas_call(
        _tc2_body,
        grid=(NP // BLK,),
        in_specs=[
            pl.BlockSpec((NC, BLK, HID), lambda i: (0, i, 0)),
            pl.BlockSpec((BLK, HID), lambda i: (i, 0)),
            pl.BlockSpec((BLK, 1), lambda i: (i, 0)),
            pl.BlockSpec((1, HID), lambda i: (0, 0)),
            pl.BlockSpec((HID, HID), lambda i: (0, 0)),
        ],
        out_specs=pl.BlockSpec((BLK, HID), lambda i: (i, 0)),
        out_shape=jax.ShapeDtypeStruct((NP, HID), jnp.float32),
    )(p1, g1, dinv, b1, w2)


def _tc3_body(p2_ref, g2_ref, dinv_ref, b2_ref, z_ref):
    ssum = p2_ref[0] + p2_ref[1] + g2_ref[...]
    z_ref[...] = ssum[:, :OUT] * dinv_ref[...] + b2_ref[...]


def _tc3(p2, g2, dinv, b2):
    return pl.pallas_call(
        _tc3_body,
        grid=(NP // BLK,),
        in_specs=[
            pl.BlockSpec((NC, BLK, HID), lambda i: (0, i, 0)),
            pl.BlockSpec((BLK, HID), lambda i: (i, 0)),
            pl.BlockSpec((BLK, 1), lambda i: (i, 0)),
            pl.BlockSpec((1, OUT), lambda i: (0, 0)),
        ],
        out_specs=pl.BlockSpec((BLK, OUT), lambda i: (i, 0)),
        out_shape=jax.ShapeDtypeStruct((NP, OUT), jnp.float32),
    )(p2, g2, dinv, b2)


def kernel(x, edge_index, W1, b1, W2, b2):
    f32 = jnp.float32
    src = edge_index[0].astype(jnp.int32)
    dst = edge_index[1].astype(jnp.int32)
    padi = jnp.full((EPAD - E,), N, jnp.int32)
    srcp = jnp.concatenate([src, padi]).reshape(NC, NS, CH, C)
    dstp = jnp.concatenate([dst, padi]).reshape(NC, NS, CH, C)
    xp = jnp.concatenate([x.astype(f32), jnp.zeros((NP - N, F_IN), f32)])

    onesc = jnp.ones((C,), f32)
    degp = _sc_degree(dstp, onesc, jnp.zeros((RPT,), f32))
    g1, dinv = _tc1(xp, W1.astype(f32), degp)
    zrow8 = jnp.zeros((RPT, HID), f32)
    p1 = _sc_agg_hid(srcp, dstp, g1, zrow8)
    w2p = jnp.zeros((HID, HID), f32).at[:, :OUT].set(W2.astype(f32))
    g2 = _tc2(p1, g1, dinv, b1.reshape(1, HID).astype(f32), w2p)
    p2 = _sc_agg_hid(srcp, dstp, g2, zrow8)
    zp = _tc3(p2, g2, dinv, b2.reshape(1, OUT).astype(f32))
    return zp[:N]


# speedup vs baseline: 45.0895x; 1.0163x over previous
import functools

import jax
import jax.numpy as jnp
from jax import lax
from jax.experimental import pallas as pl
from jax.experimental.pallas import tpu as pltpu
from jax.experimental.pallas import tpu_sc as plsc

N = 10000
E = 320000
F_IN = 128
HID = 8
OUT = 2

NC = 2
NS = 16
C = 128
CH = 80
R = 4
NG = CH // R
EPT = CH * C
EPAD = NC * NS * EPT
NP = 10240
RPT = NP // NS
BLK = 1024


def _mesh():
    return plsc.VectorSubcoreMesh(
        core_axis_name="c", subcore_axis_name="s",
        num_cores=NC, num_subcores=NS)


@functools.partial(
    pl.kernel,
    out_type=jax.ShapeDtypeStruct((NC, NP), jnp.float32),
    mesh=_mesh(),
    compiler_params=pltpu.CompilerParams(use_tc_tiling_on_sc=False),
    scratch_types=[
        pltpu.VMEM((CH, C), jnp.int32),
        pltpu.VMEM((C,), jnp.float32),
        pltpu.VMEM((RPT,), jnp.float32),
        pltpu.VMEM_SHARED((NP,), jnp.float32),
    ],
)
def _sc_degree(dstp, onesc, zrow, out, idx_v, ones_v, row_v, acc_sh):
    c = lax.axis_index("c")
    s = lax.axis_index("s")
    pltpu.sync_copy(dstp.at[c, s], idx_v)
    pltpu.sync_copy(onesc, ones_v)
    pltpu.sync_copy(zrow, row_v)
    pltpu.sync_copy(row_v, acc_sh.at[pl.ds(s * RPT, RPT)])
    plsc.subcore_barrier()

    def body(j, carry):
        pltpu.sync_copy(ones_v, acc_sh.at[idx_v.at[j]], add=True)
        return carry

    lax.fori_loop(0, CH, body, 0)
    plsc.subcore_barrier()
    pltpu.sync_copy(acc_sh.at[pl.ds(s * RPT, RPT)], row_v)
    pltpu.sync_copy(row_v, out.at[c, pl.ds(s * RPT, RPT)])


def _make_agg(D):
    @functools.partial(
        pl.kernel,
        out_type=jax.ShapeDtypeStruct((NC, NP, D), jnp.float32),
        mesh=_mesh(),
        compiler_params=pltpu.CompilerParams(use_tc_tiling_on_sc=False),
        scratch_types=[
            pltpu.VMEM((CH, C), jnp.int32),
            pltpu.VMEM((CH, C), jnp.int32),
            pltpu.VMEM((R, C, D), jnp.float32),
            pltpu.VMEM((RPT, D), jnp.float32),
            pltpu.VMEM_SHARED((NP, D), jnp.float32),
            pltpu.SemaphoreType.DMA,
            pltpu.SemaphoreType.DMA,
            pltpu.SemaphoreType.DMA,
            pltpu.SemaphoreType.DMA,
        ],
    )
    def agg(srcp, dstp, g, zrow, out,
            src_v, dst_v, rows_v, buf_v, acc_sh, s0, s1, s2, s3):
        sems = (s0, s1, s2, s3)
        c = lax.axis_index("c")
        s = lax.axis_index("s")
        pltpu.sync_copy(srcp.at[c, s], src_v)
        pltpu.sync_copy(dstp.at[c, s], dst_v)
        for b in range(R):
            pltpu.async_copy(g.at[src_v.at[b]], rows_v.at[b], sems[b])
        pltpu.sync_copy(zrow, buf_v)
        pltpu.sync_copy(buf_v, acc_sh.at[pl.ds(s * RPT, RPT)])
        plsc.subcore_barrier()

        def body(gi, carry):
            for b in range(R):
                j = gi * R + b
                pltpu.make_async_copy(
                    g.at[src_v.at[b]], rows_v.at[b], sems[b]).wait()
                pltpu.sync_copy(rows_v.at[b], acc_sh.at[dst_v.at[j]],
                                add=True)
                pltpu.async_copy(g.at[src_v.at[j + R]], rows_v.at[b],
                                 sems[b])
            return carry

        lax.fori_loop(0, NG - 1, body, 0)
        for b in range(R):
            j = (NG - 1) * R + b
            pltpu.make_async_copy(
                g.at[src_v.at[b]], rows_v.at[b], sems[b]).wait()
            pltpu.sync_copy(rows_v.at[b], acc_sh.at[dst_v.at[j]], add=True)
        plsc.subcore_barrier()
        pltpu.sync_copy(acc_sh.at[pl.ds(s * RPT, RPT)], buf_v)
        pltpu.sync_copy(buf_v, out.at[c, pl.ds(s * RPT, RPT)])

    return agg


_sc_agg_hid = _make_agg(HID)


def _tc1_body(xp_ref, w1_ref, degp_ref, g1_ref, dinv_ref):
    deg = degp_ref[0, :] + degp_ref[1, :] + 1.0
    dinv = lax.rsqrt(deg)[:, None]
    h = jnp.dot(xp_ref[...], w1_ref[...],
                preferred_element_type=jnp.float32)
    g1_ref[...] = h * dinv
    dinv_ref[...] = dinv


def _tc1(xp, w1, degp):
    return pl.pallas_call(
        _tc1_body,
        grid=(NP // BLK,),
        in_specs=[
            pl.BlockSpec((BLK, F_IN), lambda i: (i, 0)),
            pl.BlockSpec((F_IN, HID), lambda i: (0, 0)),
            pl.BlockSpec((NC, BLK), lambda i: (0, i)),
        ],
        out_specs=[
            pl.BlockSpec((BLK, HID), lambda i: (i, 0)),
            pl.BlockSpec((BLK, 1), lambda i: (i, 0)),
        ],
        out_shape=[
            jax.ShapeDtypeStruct((NP, HID), jnp.float32),
            jax.ShapeDtypeStruct((NP, 1), jnp.float32),
        ],
    )(xp, w1, degp)


def _tc2_body(p1_ref, g1_ref, dinv_ref, b1_ref, w2_ref, g2_ref):
    ssum = p1_ref[0] + p1_ref[1] + g1_ref[...]
    h = jnp.maximum(ssum * dinv_ref[...] + b1_ref[...], 0.0)
    h2 = jnp.dot(h, w2_ref[...], preferred_element_type=jnp.float32)
    g2_ref[...] = h2 * dinv_ref[...]


def _tc2(p1, g1, dinv, b1, w2):
    return pl.pallas_call(
        _tc2_body,
        grid=(NP // BLK,),
        in_specs=[
            pl.BlockSpec((NC, BLK, HID), lambda i: (0, i, 0)),
            pl.BlockSpec((BLK, HID), lambda i: (i, 0)),
            pl.BlockSpec((BLK, 1), lambda i: (i, 0)),
            pl.BlockSpec((1, HID), lambda i: (0, 0)),
            pl.BlockSpec((HID, HID), lambda i: (0, 0)),
        ],
        out_specs=pl.BlockSpec((BLK, HID), lambda i: (i, 0)),
        out_shape=jax.ShapeDtypeStruct((NP, HID), jnp.float32),
    )(p1, g1, dinv, b1, w2)


def _tc3_body(p2_ref, g2_ref, dinv_ref, b2_ref, z_ref):
    ssum = p2_ref[0] + p2_ref[1] + g2_ref[...]
    z_ref[...] = ssum[:, :OUT] * dinv_ref[...] + b2_ref[...]


def _tc3(p2, g2, dinv, b2):
    return pl.pallas_call(
        _tc3_body,
        grid=(NP // BLK,),
        in_specs=[
            pl.BlockSpec((NC, BLK, HID), lambda i: (0, i, 0)),
            pl.BlockSpec((BLK, HID), lambda i: (i, 0)),
            pl.BlockSpec((BLK, 1), lambda i: (i, 0)),
            pl.BlockSpec((1, OUT), lambda i: (0, 0)),
        ],
        out_specs=pl.BlockSpec((BLK, OUT), lambda i: (i, 0)),
        out_shape=jax.ShapeDtypeStruct((NP, OUT), jnp.float32),
    )(p2, g2, dinv, b2)


def kernel(x, edge_index, W1, b1, W2, b2):
    f32 = jnp.float32
    src = edge_index[0].astype(jnp.int32)
    dst = edge_index[1].astype(jnp.int32)
    npad = EPAD - E
    padi = jnp.full((npad,), N, jnp.int32)
    padd = N + jnp.arange(npad, dtype=jnp.int32) % (NP - N)
    srcp = jnp.concatenate([src, padi]).reshape(NC, NS, CH, C)
    dstp = jnp.concatenate([dst, padd]).reshape(NC, NS, CH, C)
    xp = jnp.concatenate([x.astype(f32), jnp.zeros((NP - N, F_IN), f32)])

    onesc = jnp.ones((C,), f32)
    degp = _sc_degree(dstp, onesc, jnp.zeros((RPT,), f32))
    g1, dinv = _tc1(xp, W1.astype(f32), degp)
    zrow8 = jnp.zeros((RPT, HID), f32)
    p1 = _sc_agg_hid(srcp, dstp, g1, zrow8)
    w2p = jnp.zeros((HID, HID), f32).at[:, :OUT].set(W2.astype(f32))
    g2 = _tc2(p1, g1, dinv, b1.reshape(1, HID).astype(f32), w2p)
    p2 = _sc_agg_hid(srcp, dstp, g2, zrow8)
    zp = _tc3(p2, g2, dinv, b2.reshape(1, OUT).astype(f32))
    return zp[:N]
